# baseline (speedup 1.0000x reference)
import functools
import jax
import jax.numpy as jnp
from jax.experimental import pallas as pl
from jax.experimental.pallas import tpu as pltpu

_N = 10000
_E = 160000
_DF = 256
_H1 = 256
_EMB = 128
_AUG = 4
_NEG = 5
_MB = 2000


def _softplus(z):
    return jnp.log(1.0 + jnp.exp(-jnp.abs(z))) + jnp.maximum(z, 0.0)


def _mm_body(a_ref, w_ref, o_ref):
    o_ref[...] = jnp.dot(a_ref[...], w_ref[...],
                         preferred_element_type=jnp.float32)


def _mm(a, w):
    m, k = a.shape
    n = w.shape[1]
    grid = m // _MB
    return pl.pallas_call(
        _mm_body,
        grid=(grid,),
        in_specs=[pl.BlockSpec((_MB, k), lambda i: (i, 0)),
                  pl.BlockSpec((k, n), lambda i: (0, 0))],
        out_specs=pl.BlockSpec((_MB, n), lambda i: (i, 0)),
        out_shape=jax.ShapeDtypeStruct((m, n), jnp.float32),
    )(a, w)


def _mm_relu_body(a_ref, b_ref, w_ref, o_ref):
    h = jnp.maximum(a_ref[...] + b_ref[...], 0.0)
    o_ref[...] = jnp.dot(h, w_ref[...], preferred_element_type=jnp.float32)


def _mm_relu(a, b, w):
    m, k = a.shape
    n = w.shape[1]
    grid = m // _MB
    return pl.pallas_call(
        _mm_relu_body,
        grid=(grid,),
        in_specs=[pl.BlockSpec((_MB, k), lambda i: (i, 0)),
                  pl.BlockSpec((1, k), lambda i: (0, 0)),
                  pl.BlockSpec((k, n), lambda i: (0, 0))],
        out_specs=pl.BlockSpec((_MB, n), lambda i: (i, 0)),
        out_shape=jax.ShapeDtypeStruct((m, n), jnp.float32),
    )(a, b.reshape(1, k), w)


def _colsum_body(h_ref, o_ref):
    i = pl.program_id(0)

    @pl.when(i == 0)
    def _():
        o_ref[...] = jnp.zeros_like(o_ref)

    o_ref[...] += jnp.sum(h_ref[...], axis=0, keepdims=True)

    @pl.when(i == pl.num_programs(0) - 1)
    def _():
        o_ref[...] = jax.nn.sigmoid(o_ref[...] / _N)


def _readout(h):
    return pl.pallas_call(
        _colsum_body,
        grid=(_N // _MB,),
        in_specs=[pl.BlockSpec((_MB, _EMB), lambda i: (i, 0))],
        out_specs=pl.BlockSpec((1, _EMB), lambda i: (0, 0)),
        out_shape=jax.ShapeDtypeStruct((1, _EMB), jnp.float32),
    )(h)


def _tail_body(h_ref, ch_ref, an_ref, c_ref, bi_ref, d1h_ref, d1n_ref,
               db1_ref, a1_ref, d2_ref, db2_ref, a2_ref, w_ref,
               ah_ref, s_ref):
    i = pl.program_id(0)
    v = jnp.dot(bi_ref[...], c_ref[...].T,
                preferred_element_type=jnp.float32)
    h = h_ref[...]
    s_h = jnp.sum(_softplus(-jnp.dot(h, v, preferred_element_type=jnp.float32)))
    s_corp = jnp.sum(_softplus(jnp.dot(ch_ref[...], v,
                                       preferred_element_type=jnp.float32)))
    z = (jnp.dot(h, d1h_ref[...], preferred_element_type=jnp.float32)
         + jnp.dot(an_ref[...], d1n_ref[...], preferred_element_type=jnp.float32)
         + db1_ref[...])
    z = jnp.maximum(z, 0.0) + a1_ref[...] * jnp.minimum(z, 0.0)
    ah = jnp.dot(z, d2_ref[...], preferred_element_type=jnp.float32) + db2_ref[...]
    ah = jnp.maximum(ah, 0.0) + a2_ref[...] * jnp.minimum(ah, 0.0)
    ah_ref[...] = ah
    s_ah = jnp.sum(_softplus(-jnp.dot(ah, v, preferred_element_type=jnp.float32)))
    p = jnp.sum(jnp.abs(ah - h) * w_ref[...], axis=1)
    s_pos = jnp.sum(_softplus(-p))

    @pl.when(i == 0)
    def _():
        s_ref[...] = jnp.zeros_like(s_ref)

    lane = jax.lax.broadcasted_iota(jnp.int32, (1, 128), 1)
    s_ref[...] += (jnp.where(lane == 0, s_h, 0.0)
                   + jnp.where(lane == 1, s_corp, 0.0)
                   + jnp.where(lane == 2, s_ah, 0.0)
                   + jnp.where(lane == 3, s_pos, 0.0))


def _tail(h, corp_h, aug_noise, c, bi, D1, db1, a1, D2, db2, a2, w):
    d1h = D1[:_EMB]
    d1n = D1[_EMB:]
    full = pl.BlockSpec(None, lambda i: None)
    row = lambda r, k: pl.BlockSpec((r, k), lambda i: (i, 0))
    ah, stats = pl.pallas_call(
        _tail_body,
        grid=(_N // _MB,),
        in_specs=[row(_MB, _EMB), row(_MB, _EMB), row(_MB, 16),
                  pl.BlockSpec((1, _EMB), lambda i: (0, 0)),
                  pl.BlockSpec((_EMB, _EMB), lambda i: (0, 0)),
                  pl.BlockSpec((_EMB, 256), lambda i: (0, 0)),
                  pl.BlockSpec((16, 256), lambda i: (0, 0)),
                  pl.BlockSpec((1, 256), lambda i: (0, 0)),
                  pl.BlockSpec((1, 256), lambda i: (0, 0)),
                  pl.BlockSpec((256, _EMB), lambda i: (0, 0)),
                  pl.BlockSpec((1, _EMB), lambda i: (0, 0)),
                  pl.BlockSpec((1, _EMB), lambda i: (0, 0)),
                  pl.BlockSpec((1, _EMB), lambda i: (0, 0))],
        out_specs=[pl.BlockSpec((_MB, _EMB), lambda i: (i, 0)),
                   pl.BlockSpec((1, 128), lambda i: (0, 0))],
        out_shape=[jax.ShapeDtypeStruct((_N, _EMB), jnp.float32),
                   jax.ShapeDtypeStruct((1, 128), jnp.float32)],
    )(h, corp_h, aug_noise, c, bi, d1h, d1n, db1.reshape(1, -1),
      a1.reshape(1, -1), D2, db2.reshape(1, -1), a2.reshape(1, -1),
      w.reshape(1, -1))
    return ah, stats


def _neg_body(q_ref, o_ref):
    i = pl.program_id(0)

    @pl.when(i == 0)
    def _():
        o_ref[...] = jnp.zeros_like(o_ref)

    o_ref[...] += jnp.zeros((1, 128), jnp.float32) + jnp.sum(_softplus(q_ref[...]))


def _neg_sum(q):
    m = q.shape[0]
    blk = m // 7
    return pl.pallas_call(
        _neg_body,
        grid=(7,),
        in_specs=[pl.BlockSpec((blk, q.shape[1]), lambda i: (i, 0))],
        out_specs=pl.BlockSpec((1, 128), lambda i: (0, 0)),
        out_shape=jax.ShapeDtypeStruct((1, 128), jnp.float32),
    )(q)


def kernel(x, edge_index, edge_weight, corp_x, corp_edge_index, corp_edge_weight,
           negative_index, aug_noise, W1, b1, W2, b2, D1, db1, a1, D2, db2, a2,
           bi_weights, siamese_w):
    src, dst = edge_index[0], edge_index[1]
    csrc, cdst = corp_edge_index[0], corp_edge_index[1]

    x2 = jnp.concatenate([x, corp_x], axis=0)
    xw = _mm(x2, W1)

    agg1 = jax.ops.segment_sum(xw[:_N][src] * edge_weight[:, None], dst,
                               num_segments=_N)
    agg1c = jax.ops.segment_sum(xw[_N:][csrc] * corp_edge_weight[:, None], cdst,
                                num_segments=_N)
    hw = _mm_relu(jnp.concatenate([agg1, agg1c], axis=0), b1, W2)

    agg2 = jax.ops.segment_sum(hw[:_N][src] * edge_weight[:, None], dst,
                               num_segments=_N)
    agg2c = jax.ops.segment_sum(hw[_N:][csrc] * corp_edge_weight[:, None], cdst,
                                num_segments=_N)
    h = jax.nn.relu(agg2 + b2)
    corp_h = jax.nn.relu(agg2c + b2)

    c = _readout(h)
    ah, stats = _tail(h, corp_h, aug_noise, c, bi_weights,
                      D1, db1, a1, D2, db2, a2, siamese_w[0])
    s_h, s_corp, s_ah, s_pos = stats[0, 0], stats[0, 1], stats[0, 2], stats[0, 3]

    neg = h[negative_index.reshape(-1)]
    ah_rep = jnp.tile(ah, (_AUG, 1))
    ah_rep = jnp.repeat(ah_rep, _NEG, axis=0)
    q = jnp.sum(jnp.abs(ah_rep - neg) * siamese_w[0], axis=-1)
    qp = jnp.pad(q, (0, 704), constant_values=-1e30).reshape(1568, 128)
    s_neg = _neg_sum(qp)[0, 0]

    dgi = (s_h + s_corp) / (2.0 * _N)
    aug_dgi = _AUG * 1e-05 * (s_ah + s_corp) / (2.0 * _N)
    siam = 1e-05 * (_AUG * 10.0 * s_pos + s_neg) / ((_AUG + _AUG * _NEG) * _N)
    return dgi + aug_dgi + siam


# speedup vs baseline: 1.6373x; 1.6373x over previous
import functools
import jax
import jax.numpy as jnp
from jax import lax
from jax.experimental import pallas as pl
from jax.experimental.pallas import tpu as pltpu
from jax.experimental.pallas import tpu_sc as plsc

_N = 10000
_E = 160000
_EPAD = 163840
_MB = 2000
_AUG = 4
_NEG = 5


def _softplus(z):
    return jnp.log(1.0 + jnp.exp(-jnp.abs(z))) + jnp.maximum(z, 0.0)


def _mesh():
    return plsc.VectorSubcoreMesh(core_axis_name="c", subcore_axis_name="s")


def _make_seg(layer):
    chunks = 80 if layer == 1 else 40

    @functools.partial(
        pl.kernel,
        out_type=jax.ShapeDtypeStruct((4 * _N, 128), jnp.float32),
        mesh=_mesh(),
        scratch_types=[
            pltpu.VMEM((128,), jnp.int32),
            pltpu.VMEM((128,), jnp.int32),
            pltpu.VMEM((128, 16), jnp.float32),
            pltpu.VMEM((128, 128), jnp.float32),
            pltpu.VMEM_SHARED((_N, 128), jnp.float32),
            pltpu.SemaphoreType.DMA,
        ],
    )
    def seg(src1, dst1, ew1, src2, dst2, ew2, table, zeros_hbm, out,
            src_v, dst_v, ewb, rows_v, acc, sem):
        c = lax.axis_index("c")
        s = lax.axis_index("s")

        if layer == 1:
            base = s * (chunks * 128)
            toff = c * (2 * _N)
        else:
            base = (c * 16 + s) * (chunks * 128)
            toff = c * 0

        for g, (srcr, dstr, ewr) in enumerate(((src1, dst1, ew1),
                                               (src2, dst2, ew2))):
            @pl.when(s < 15)
            def _():
                pltpu.sync_copy(zeros_hbm, acc.at[pl.ds(s * 632, 632)])

            @pl.when(s == 15)
            def _():
                pltpu.sync_copy(zeros_hbm.at[pl.ds(0, 520)],
                                acc.at[pl.ds(15 * 632, 520)])

            plsc.subcore_barrier()

            def chunk_body(j, carry):
                e0 = base + j * 128
                pltpu.sync_copy(srcr.at[pl.ds(e0, 128)], src_v)
                pltpu.sync_copy(dstr.at[pl.ds(e0, 128)], dst_v)
                pltpu.sync_copy(ewr.at[pl.ds(e0, 128)], ewb)
                if layer == 1:
                    for b in range(8):
                        src_v[pl.ds(b * 16, 16)] = src_v[pl.ds(b * 16, 16)] + toff
                pltpu.async_copy(table.at[src_v], rows_v, sem).wait()

                def srow(r, carry2):
                    wspl = ewb[r]
                    for cb in range(8):
                        rows_v[r, pl.ds(cb * 16, 16)] = (
                            rows_v[r, pl.ds(cb * 16, 16)] * wspl)
                    return carry2

                lax.fori_loop(0, 128, srow, 0)
                pltpu.sync_copy(rows_v, acc.at[dst_v], add=True)
                return carry

            lax.fori_loop(0, chunks, chunk_body, 0)
            plsc.subcore_barrier()

            @pl.when(s < 15)
            def _():
                pltpu.sync_copy(
                    acc.at[pl.ds(s * 632, 632)],
                    out.at[pl.ds(g * 2 * _N + c * _N + s * 632, 632)])

            @pl.when(s == 15)
            def _():
                pltpu.sync_copy(
                    acc.at[pl.ds(15 * 632, 520)],
                    out.at[pl.ds(g * 2 * _N + c * _N + 15 * 632, 520)])

            plsc.subcore_barrier()

    return seg


_seg1 = _make_seg(1)
_seg2 = _make_seg(2)


@functools.partial(
    pl.kernel,
    out_type=jax.ShapeDtypeStruct((1600, 125, 16), jnp.float32),
    mesh=_mesh(),
    scratch_types=[
        pltpu.VMEM((50, 125), jnp.int32),
        pltpu.VMEM((125, 128), jnp.float32),
        pltpu.VMEM((32, 128), jnp.float32),
        pltpu.VMEM((128,), jnp.float32),
        pltpu.VMEM((125, 16), jnp.float32),
        pltpu.SemaphoreType.DMA,
    ],
)
def _neg(h_tab, ah, nidx3, w_hbm, out, idx_v, negbuf, ah_v, w_v, obuf, sem):
    c = lax.axis_index("c")
    s = lax.axis_index("s")
    wid = c * 16 + s
    pltpu.sync_copy(w_hbm, w_v)
    pltpu.sync_copy(nidx3.at[wid], idx_v)
    kbase = wid * 1250

    def chunk(j, carry):
        i0 = lax.rem(kbase + j * 25, _N)
        i0a = (i0 // 8) * 8
        off = i0 - i0a
        pltpu.sync_copy(ah.at[pl.ds(i0a, 32)], ah_v)
        pltpu.async_copy(h_tab.at[idx_v.at[j]], negbuf, sem).wait()

        def kbody(kl, carry2):
            a = [ah_v[kl + off, pl.ds(cb * 16, 16)] for cb in range(8)]
            wv = [w_v[pl.ds(cb * 16, 16)] for cb in range(8)]
            for jn in range(_NEG):
                r = kl * _NEG + jn
                t = jnp.zeros((16,), jnp.float32)
                for cb in range(8):
                    t = t + jnp.abs(negbuf[r, pl.ds(cb * 16, 16)] - a[cb]) * wv[cb]
                obuf[r] = t
            return carry2

        lax.fori_loop(0, 25, kbody, 0)
        pltpu.sync_copy(obuf, out.at[wid * 50 + j])
        return carry

    lax.fori_loop(0, 50, chunk, 0)


def _mm_split_body(a_ref, w_ref, o_ref):
    o_ref[...] = jnp.dot(a_ref[...], w_ref[...],
                         preferred_element_type=jnp.float32)


def _mm_split(a, w):
    m = a.shape[0]
    nb = m // _MB
    return pl.pallas_call(
        _mm_split_body,
        grid=(nb, 2),
        in_specs=[pl.BlockSpec((_MB, 256), lambda i, jc: (i, 0)),
                  pl.BlockSpec((256, 128), lambda i, jc: (0, jc))],
        out_specs=pl.BlockSpec((_MB, 128), lambda i, jc: (jc * nb + i, 0)),
        out_shape=jax.ShapeDtypeStruct((2 * m, 128), jnp.float32),
    )(a, w)


def _mm_relu2_body(al_ref, ar_ref, bl_ref, br_ref, wl_ref, wr_ref, o_ref):
    hl = jnp.maximum(al_ref[...] + bl_ref[...], 0.0)
    hr = jnp.maximum(ar_ref[...] + br_ref[...], 0.0)
    o_ref[...] = (jnp.dot(hl, wl_ref[...], preferred_element_type=jnp.float32)
                  + jnp.dot(hr, wr_ref[...], preferred_element_type=jnp.float32))


def _mm_relu2(ag, b1, w2):
    bl = b1[:128].reshape(1, 128)
    br = b1[128:].reshape(1, 128)
    wl = w2[:128]
    wr = w2[128:]
    return pl.pallas_call(
        _mm_relu2_body,
        grid=(5, 2),
        in_specs=[pl.BlockSpec((_MB, 128), lambda i, g: (g * 10 + i, 0)),
                  pl.BlockSpec((_MB, 128), lambda i, g: (g * 10 + 5 + i, 0)),
                  pl.BlockSpec((1, 128), lambda i, g: (0, 0)),
                  pl.BlockSpec((1, 128), lambda i, g: (0, 0)),
                  pl.BlockSpec((128, 128), lambda i, g: (0, 0)),
                  pl.BlockSpec((128, 128), lambda i, g: (0, 0))],
        out_specs=pl.BlockSpec((_MB, 128), lambda i, g: (g * 5 + i, 0)),
        out_shape=jax.ShapeDtypeStruct((2 * _N, 128), jnp.float32),
    )(ag, ag, bl, br, wl, wr)


def _final_h_body(p0_ref, p1_ref, b_ref, o_ref):
    o_ref[...] = jnp.maximum(p0_ref[...] + p1_ref[...] + b_ref[...], 0.0)


def _final_h(sg2, b2):
    return pl.pallas_call(
        _final_h_body,
        grid=(5, 2),
        in_specs=[pl.BlockSpec((_MB, 128), lambda i, g: (g * 10 + i, 0)),
                  pl.BlockSpec((_MB, 128), lambda i, g: (g * 10 + 5 + i, 0)),
                  pl.BlockSpec((1, 128), lambda i, g: (0, 0))],
        out_specs=pl.BlockSpec((_MB, 128), lambda i, g: (g * 5 + i, 0)),
        out_shape=jax.ShapeDtypeStruct((2 * _N, 128), jnp.float32),
    )(sg2, sg2, b2.reshape(1, 128))


def _colsum_body(h_ref, o_ref):
    i = pl.program_id(0)

    @pl.when(i == 0)
    def _():
        o_ref[...] = jnp.zeros_like(o_ref)

    o_ref[...] += jnp.sum(h_ref[...], axis=0, keepdims=True)

    @pl.when(i == pl.num_programs(0) - 1)
    def _():
        o_ref[...] = jax.nn.sigmoid(o_ref[...] / _N)


def _readout(h2):
    return pl.pallas_call(
        _colsum_body,
        grid=(5,),
        in_specs=[pl.BlockSpec((_MB, 128), lambda i: (i, 0))],
        out_specs=pl.BlockSpec((1, 128), lambda i: (0, 0)),
        out_shape=jax.ShapeDtypeStruct((1, 128), jnp.float32),
    )(h2)


def _tail_body(h_ref, ch_ref, an_ref, c_ref, bi_ref, d1h_ref, d1n_ref,
               db1_ref, a1_ref, d2_ref, db2_ref, a2_ref, w_ref,
               ah_ref, s_ref):
    i = pl.program_id(0)
    v = jnp.dot(bi_ref[...], c_ref[...].T,
                preferred_element_type=jnp.float32)
    h = h_ref[...]
    s_h = jnp.sum(_softplus(-jnp.dot(h, v, preferred_element_type=jnp.float32)))
    s_corp = jnp.sum(_softplus(jnp.dot(ch_ref[...], v,
                                       preferred_element_type=jnp.float32)))
    z = (jnp.dot(h, d1h_ref[...], preferred_element_type=jnp.float32)
         + jnp.dot(an_ref[...], d1n_ref[...], preferred_element_type=jnp.float32)
         + db1_ref[...])
    z = jnp.maximum(z, 0.0) + a1_ref[...] * jnp.minimum(z, 0.0)
    ah = jnp.dot(z, d2_ref[...], preferred_element_type=jnp.float32) + db2_ref[...]
    ah = jnp.maximum(ah, 0.0) + a2_ref[...] * jnp.minimum(ah, 0.0)
    ah_ref[...] = ah
    s_ah = jnp.sum(_softplus(-jnp.dot(ah, v, preferred_element_type=jnp.float32)))
    p = jnp.sum(jnp.abs(ah - h) * w_ref[...], axis=1)
    s_pos = jnp.sum(_softplus(-p))

    @pl.when(i == 0)
    def _():
        s_ref[...] = jnp.zeros_like(s_ref)

    lane = jax.lax.broadcasted_iota(jnp.int32, (1, 128), 1)
    s_ref[...] += (jnp.where(lane == 0, s_h, 0.0)
                   + jnp.where(lane == 1, s_corp, 0.0)
                   + jnp.where(lane == 2, s_ah, 0.0)
                   + jnp.where(lane == 3, s_pos, 0.0))


def _tail(h2, aug_noise, c, bi, D1, db1, a1, D2, db2, a2, w):
    d1h = D1[:128]
    d1n = D1[128:]
    ah, stats = pl.pallas_call(
        _tail_body,
        grid=(5,),
        in_specs=[pl.BlockSpec((_MB, 128), lambda i: (i, 0)),
                  pl.BlockSpec((_MB, 128), lambda i: (i + 5, 0)),
                  pl.BlockSpec((_MB, 16), lambda i: (i, 0)),
                  pl.BlockSpec((1, 128), lambda i: (0, 0)),
                  pl.BlockSpec((128, 128), lambda i: (0, 0)),
                  pl.BlockSpec((128, 256), lambda i: (0, 0)),
                  pl.BlockSpec((16, 256), lambda i: (0, 0)),
                  pl.BlockSpec((1, 256), lambda i: (0, 0)),
                  pl.BlockSpec((1, 256), lambda i: (0, 0)),
                  pl.BlockSpec((256, 128), lambda i: (0, 0)),
                  pl.BlockSpec((1, 128), lambda i: (0, 0)),
                  pl.BlockSpec((1, 128), lambda i: (0, 0)),
                  pl.BlockSpec((1, 128), lambda i: (0, 0))],
        out_specs=[pl.BlockSpec((_MB, 128), lambda i: (i, 0)),
                   pl.BlockSpec((1, 128), lambda i: (0, 0))],
        out_shape=[jax.ShapeDtypeStruct((_N, 128), jnp.float32),
                   jax.ShapeDtypeStruct((1, 128), jnp.float32)],
    )(h2, h2, aug_noise, c, bi, d1h, d1n, db1.reshape(1, -1),
      a1.reshape(1, -1), D2, db2.reshape(1, -1), a2.reshape(1, -1),
      w.reshape(1, -1))
    return ah, stats


def _negsum_body(t_ref, o_ref):
    i = pl.program_id(0)

    @pl.when(i == 0)
    def _():
        o_ref[...] = jnp.zeros_like(o_ref)

    blk = t_ref[...]
    d = jax.lax.broadcasted_iota(jnp.int32, (128, 8), 0)
    g = jax.lax.broadcasted_iota(jnp.int32, (128, 8), 1)
    gmat = (d // 16 == g).astype(jnp.float32)
    q = jnp.dot(blk, gmat, preferred_element_type=jnp.float32)
    o_ref[...] += jnp.zeros((1, 128), jnp.float32) + jnp.sum(_softplus(q))


def _negsum(t):
    return pl.pallas_call(
        _negsum_body,
        grid=(25,),
        in_specs=[pl.BlockSpec((1000, 128), lambda i: (i, 0))],
        out_specs=pl.BlockSpec((1, 128), lambda i: (0, 0)),
        out_shape=jax.ShapeDtypeStruct((1, 128), jnp.float32),
    )(t)


def kernel(x, edge_index, edge_weight, corp_x, corp_edge_index, corp_edge_weight,
           negative_index, aug_noise, W1, b1, W2, b2, D1, db1, a1, D2, db2, a2,
           bi_weights, siamese_w):
    src = edge_index[0].astype(jnp.int32)
    dst = edge_index[1].astype(jnp.int32)
    csrc = corp_edge_index[0].astype(jnp.int32)
    cdst = corp_edge_index[1].astype(jnp.int32)
    pad = _EPAD - _E
    src1 = jnp.pad(src, (0, pad))
    dst1 = jnp.pad(dst, (0, pad))
    ew1 = jnp.broadcast_to(jnp.pad(edge_weight, (0, pad))[:, None],
                           (_EPAD, 16))
    src2 = jnp.pad(csrc + _N, (0, pad))
    dst2 = jnp.pad(cdst, (0, pad))
    ew2 = jnp.broadcast_to(jnp.pad(corp_edge_weight, (0, pad))[:, None],
                           (_EPAD, 16))

    zeros_hbm = jnp.zeros((632, 128), jnp.float32)
    xs = jnp.concatenate([x, corp_x], axis=0)
    xw = _mm_split(xs, W1)
    ag = _seg1(src1, dst1, ew1, src2, dst2, ew2, xw, zeros_hbm)
    hw = _mm_relu2(ag, b1, W2)
    sg2 = _seg2(src1, dst1, ew1, src2, dst2, ew2, hw, zeros_hbm)
    h2 = _final_h(sg2, b2)

    c = _readout(h2)
    ah, stats = _tail(h2, aug_noise, c, bi_weights,
                      D1, db1, a1, D2, db2, a2, siamese_w[0])
    s_h, s_corp, s_ah, s_pos = stats[0, 0], stats[0, 1], stats[0, 2], stats[0, 3]

    nidx3 = negative_index.astype(jnp.int32).reshape(32, 50, 125)
    t = _neg(h2, ah, nidx3, siamese_w[0])
    s_neg = _negsum(t.reshape(25000, 128))[0, 0]

    dgi = (s_h + s_corp) / (2.0 * _N)
    aug_dgi = _AUG * 1e-05 * (s_ah + s_corp) / (2.0 * _N)
    siam = 1e-05 * (_AUG * 10.0 * s_pos + s_neg) / ((_AUG + _AUG * _NEG) * _N)
    return dgi + aug_dgi + siam


# speedup vs baseline: 1.9528x; 1.1927x over previous
import functools
import jax
import jax.numpy as jnp
from jax import lax
from jax.experimental import pallas as pl
from jax.experimental.pallas import tpu as pltpu
from jax.experimental.pallas import tpu_sc as plsc

_N = 10000
_E = 160000
_EARR = 164096
_MB = 2000
_AUG = 4
_NEG = 5


def _softplus(z):
    return jnp.log(1.0 + jnp.exp(-jnp.abs(z))) + jnp.maximum(z, 0.0)


def _mesh():
    return plsc.VectorSubcoreMesh(core_axis_name="c", subcore_axis_name="s")


def _make_seg(q_groups):
    chunks = 80

    @functools.partial(
        pl.kernel,
        out_type=jax.ShapeDtypeStruct((q_groups * 2 * _N, 64), jnp.float32),
        mesh=_mesh(),
        compiler_params=pltpu.CompilerParams(use_tc_tiling_on_sc=False),
        scratch_types=[
            pltpu.VMEM((chunks, 128), jnp.int32),
            pltpu.VMEM((chunks, 128), jnp.int32),
            pltpu.VMEM((128, 16), jnp.float32),
            pltpu.VMEM((128, 16), jnp.float32),
            pltpu.VMEM((128, 64), jnp.float32),
            pltpu.VMEM((128, 64), jnp.float32),
            pltpu.VMEM_SHARED((_N, 64), jnp.float32),
            pltpu.SemaphoreType.DMA,
            pltpu.SemaphoreType.DMA,
        ],
    )
    def seg(src1, dst1, ew1, src2, dst2, ew2, table, zeros_hbm, out,
            src_a, dst_a, ewb0, ewb1, rows0, rows1, acc, sem0, sem1):
        c = lax.axis_index("c")
        s = lax.axis_index("s")
        cbase = s * chunks

        def scale(rows_v, ewb):
            def srow8(r8, carry2):
                for u in range(8):
                    r = r8 * 8 + u
                    wspl = ewb[r]
                    for cb in range(4):
                        rows_v[r, pl.ds(cb * 16, 16)] = (
                            rows_v[r, pl.ds(cb * 16, 16)] * wspl)
                return carry2

            lax.fori_loop(0, 16, srow8, 0)

        for g, (srcr, dstr, ewr) in enumerate(((src1, dst1, ew1),
                                               (src2, dst2, ew2))):
            for qq in range(q_groups // 2):
                q = c * (q_groups // 2) + qq
                toff = q * (2 * _N)

                pltpu.sync_copy(zeros_hbm, acc.at[pl.ds(s * 625, 625)])
                pltpu.sync_copy(srcr.at[pl.ds(cbase, chunks)], src_a)
                pltpu.sync_copy(dstr.at[pl.ds(cbase, chunks)], dst_a)

                def addoff(r, carry2):
                    for cb in range(8):
                        src_a[r, pl.ds(cb * 16, 16)] = (
                            src_a[r, pl.ds(cb * 16, 16)] + toff)
                    return carry2

                lax.fori_loop(0, chunks, addoff, 0)
                plsc.subcore_barrier()

                pltpu.sync_copy(ewr.at[pl.ds(cbase * 128, 128)], ewb0)
                pltpu.async_copy(table.at[src_a.at[0]], rows0, sem0)

                def pipe(j2, carry):
                    a = j2 * 2
                    pltpu.sync_copy(ewr.at[pl.ds((cbase + a + 1) * 128, 128)],
                                    ewb1)
                    pltpu.async_copy(table.at[src_a.at[a + 1]], rows1, sem1)
                    pltpu.make_async_copy(table.at[src_a.at[a]], rows0,
                                          sem0).wait()
                    scale(rows0, ewb0)
                    pltpu.sync_copy(rows0, acc.at[dst_a.at[a]], add=True)

                    @pl.when(a + 2 < chunks)
                    def _():
                        pltpu.sync_copy(
                            ewr.at[pl.ds((cbase + a + 2) * 128, 128)], ewb0)
                        pltpu.async_copy(table.at[src_a.at[a + 2]], rows0,
                                         sem0)

                    pltpu.make_async_copy(table.at[src_a.at[a + 1]], rows1,
                                          sem1).wait()
                    scale(rows1, ewb1)
                    pltpu.sync_copy(rows1, acc.at[dst_a.at[a + 1]], add=True)
                    return carry

                lax.fori_loop(0, chunks // 2, pipe, 0)
                plsc.subcore_barrier()
                obase = q * 2 * _N + g * _N
                pltpu.sync_copy(acc.at[pl.ds(s * 625, 625)],
                                out.at[pl.ds(obase + s * 625, 625)])
                plsc.subcore_barrier()

    return seg


_seg1 = _make_seg(4)
_seg2 = _make_seg(2)


@functools.partial(
    pl.kernel,
    out_type=jax.ShapeDtypeStruct((1600, 125, 16), jnp.float32),
    mesh=_mesh(),
    scratch_types=[
        pltpu.VMEM((50, 125), jnp.int32),
        pltpu.VMEM((125, 128), jnp.float32),
        pltpu.VMEM((125, 128), jnp.float32),
        pltpu.VMEM((32, 128), jnp.float32),
        pltpu.VMEM((32, 128), jnp.float32),
        pltpu.VMEM((128,), jnp.float32),
        pltpu.VMEM((125, 16), jnp.float32),
        pltpu.VMEM((125, 16), jnp.float32),
        pltpu.SemaphoreType.DMA,
        pltpu.SemaphoreType.DMA,
    ],
)
def _neg(h_tab, ah, nidx3, w_hbm, out, idx_v, neg0, neg1, ah0, ah1, w_v,
         ob0, ob1, sem0, sem1):
    c = lax.axis_index("c")
    s = lax.axis_index("s")
    wid = c * 16 + s
    pltpu.sync_copy(w_hbm, w_v)
    pltpu.sync_copy(nidx3.at[wid], idx_v)
    kbase = wid * 1250

    def load_ah(j, ahbuf):
        i0 = lax.rem(kbase + j * 25, _N)
        i0a = (i0 // 8) * 8
        pltpu.sync_copy(ah.at[pl.ds(i0a, 32)], ahbuf)
        return i0 - i0a

    def compute(j, negbuf, ahbuf, obuf, off):
        def kbody(kl, carry2):
            a = [ahbuf[kl + off, pl.ds(cb * 16, 16)] for cb in range(8)]
            wv = [w_v[pl.ds(cb * 16, 16)] for cb in range(8)]
            for jn in range(_NEG):
                r = kl * _NEG + jn
                t = jnp.zeros((16,), jnp.float32)
                for cb in range(8):
                    t = t + jnp.abs(negbuf[r, pl.ds(cb * 16, 16)] - a[cb]) * wv[cb]
                obuf[r] = t
            return carry2

        lax.fori_loop(0, 25, kbody, 0)
        pltpu.sync_copy(obuf, out.at[wid * 50 + j])

    load_ah(0, ah0)
    pltpu.async_copy(h_tab.at[idx_v.at[0]], neg0, sem0)

    def pipe(j2, carry):
        a = j2 * 2
        off0 = lax.rem(lax.rem(kbase + a * 25, _N), 8)
        off1 = load_ah(a + 1, ah1)
        pltpu.async_copy(h_tab.at[idx_v.at[a + 1]], neg1, sem1)
        pltpu.make_async_copy(h_tab.at[idx_v.at[a]], neg0, sem0).wait()
        compute(a, neg0, ah0, ob0, off0)

        @pl.when(a + 2 < 50)
        def _():
            load_ah(a + 2, ah0)
            pltpu.async_copy(h_tab.at[idx_v.at[a + 2]], neg0, sem0)

        pltpu.make_async_copy(h_tab.at[idx_v.at[a + 1]], neg1, sem1).wait()
        compute(a + 1, neg1, ah1, ob1, off1)
        return carry

    lax.fori_loop(0, 25, pipe, 0)


def _mm_split_body(a_ref, w_ref, o_ref):
    o_ref[...] = jnp.dot(a_ref[...], w_ref[0],
                         preferred_element_type=jnp.float32)


def _mm_split(a, w):
    m = a.shape[0]
    nb = m // _MB
    wr = w.reshape(256, 4, 64).transpose(1, 0, 2)
    return pl.pallas_call(
        _mm_split_body,
        grid=(nb, 4),
        in_specs=[pl.BlockSpec((_MB, 256), lambda i, jc: (i, 0)),
                  pl.BlockSpec((1, 256, 64), lambda i, jc: (jc, 0, 0))],
        out_specs=pl.BlockSpec((_MB, 64), lambda i, jc: (jc * nb + i, 0)),
        out_shape=jax.ShapeDtypeStruct((4 * m, 64), jnp.float32),
    )(a, wr)


def _mm_relu2_body(a0_ref, a1_ref, a2_ref, a3_ref, b_ref, w_ref, o_ref):
    acc = jnp.zeros((_MB, 64), jnp.float32)
    for qq, aref in enumerate((a0_ref, a1_ref, a2_ref, a3_ref)):
        hq = jnp.maximum(aref[...] + b_ref[qq, :][None, :], 0.0)
        acc = acc + jnp.dot(hq, w_ref[0, qq * 64:(qq + 1) * 64, :],
                            preferred_element_type=jnp.float32)
    o_ref[...] = acc


def _mm_relu2(ag, b1, w2):
    q = lambda qq: pl.BlockSpec(
        (_MB, 64), lambda i, g, hc, _qq=qq: (_qq * 10 + g * 5 + i, 0))
    w2r = w2.reshape(256, 2, 64).transpose(1, 0, 2)
    return pl.pallas_call(
        _mm_relu2_body,
        grid=(5, 2, 2),
        in_specs=[q(0), q(1), q(2), q(3),
                  pl.BlockSpec((4, 64), lambda i, g, hc: (0, 0)),
                  pl.BlockSpec((1, 256, 64), lambda i, g, hc: (hc, 0, 0))],
        out_specs=pl.BlockSpec((_MB, 64),
                               lambda i, g, hc: (hc * 10 + g * 5 + i, 0)),
        out_shape=jax.ShapeDtypeStruct((4 * _N, 64), jnp.float32),
    )(ag, ag, ag, ag, b1.reshape(4, 64), w2r)


def _final_h_body(l_ref, r_ref, b_ref, o_ref):
    hl = jnp.maximum(l_ref[...] + b_ref[0, :][None, :], 0.0)
    hr = jnp.maximum(r_ref[...] + b_ref[1, :][None, :], 0.0)
    o_ref[...] = jnp.concatenate([hl, hr], axis=1)


def _final_h(sg2, b2):
    return pl.pallas_call(
        _final_h_body,
        grid=(5, 2),
        in_specs=[pl.BlockSpec((_MB, 64), lambda i, g: (g * 5 + i, 0)),
                  pl.BlockSpec((_MB, 64), lambda i, g: (10 + g * 5 + i, 0)),
                  pl.BlockSpec((2, 64), lambda i, g: (0, 0))],
        out_specs=pl.BlockSpec((_MB, 128), lambda i, g: (g * 5 + i, 0)),
        out_shape=jax.ShapeDtypeStruct((2 * _N, 128), jnp.float32),
    )(sg2, sg2, b2.reshape(2, 64))


def _colsum_body(h_ref, o_ref):
    i = pl.program_id(0)

    @pl.when(i == 0)
    def _():
        o_ref[...] = jnp.zeros_like(o_ref)

    o_ref[...] += jnp.sum(h_ref[...], axis=0, keepdims=True)

    @pl.when(i == pl.num_programs(0) - 1)
    def _():
        o_ref[...] = jax.nn.sigmoid(o_ref[...] / _N)


def _readout(h2):
    return pl.pallas_call(
        _colsum_body,
        grid=(5,),
        in_specs=[pl.BlockSpec((_MB, 128), lambda i: (i, 0))],
        out_specs=pl.BlockSpec((1, 128), lambda i: (0, 0)),
        out_shape=jax.ShapeDtypeStruct((1, 128), jnp.float32),
    )(h2)


def _tail_body(h_ref, ch_ref, an_ref, c_ref, bi_ref, d1h_ref, d1n_ref,
               db1_ref, a1_ref, d2_ref, db2_ref, a2_ref, w_ref,
               ah_ref, s_ref):
    i = pl.program_id(0)
    v = jnp.dot(bi_ref[...], c_ref[...].T,
                preferred_element_type=jnp.float32)
    h = h_ref[...]
    s_h = jnp.sum(_softplus(-jnp.dot(h, v, preferred_element_type=jnp.float32)))
    s_corp = jnp.sum(_softplus(jnp.dot(ch_ref[...], v,
                                       preferred_element_type=jnp.float32)))
    z = (jnp.dot(h, d1h_ref[...], preferred_element_type=jnp.float32)
         + jnp.dot(an_ref[...], d1n_ref[...], preferred_element_type=jnp.float32)
         + db1_ref[...])
    z = jnp.maximum(z, 0.0) + a1_ref[...] * jnp.minimum(z, 0.0)
    ah = jnp.dot(z, d2_ref[...], preferred_element_type=jnp.float32) + db2_ref[...]
    ah = jnp.maximum(ah, 0.0) + a2_ref[...] * jnp.minimum(ah, 0.0)
    ah_ref[...] = ah
    s_ah = jnp.sum(_softplus(-jnp.dot(ah, v, preferred_element_type=jnp.float32)))
    p = jnp.sum(jnp.abs(ah - h) * w_ref[...], axis=1)
    s_pos = jnp.sum(_softplus(-p))

    @pl.when(i == 0)
    def _():
        s_ref[...] = jnp.zeros_like(s_ref)

    lane = jax.lax.broadcasted_iota(jnp.int32, (1, 128), 1)
    s_ref[...] += (jnp.where(lane == 0, s_h, 0.0)
                   + jnp.where(lane == 1, s_corp, 0.0)
                   + jnp.where(lane == 2, s_ah, 0.0)
                   + jnp.where(lane == 3, s_pos, 0.0))


def _tail(h2, aug_noise, c, bi, D1, db1, a1, D2, db2, a2, w):
    d1h = D1[:128]
    d1n = D1[128:]
    ah, stats = pl.pallas_call(
        _tail_body,
        grid=(5,),
        in_specs=[pl.BlockSpec((_MB, 128), lambda i: (i, 0)),
                  pl.BlockSpec((_MB, 128), lambda i: (i + 5, 0)),
                  pl.BlockSpec((_MB, 16), lambda i: (i, 0)),
                  pl.BlockSpec((1, 128), lambda i: (0, 0)),
                  pl.BlockSpec((128, 128), lambda i: (0, 0)),
                  pl.BlockSpec((128, 256), lambda i: (0, 0)),
                  pl.BlockSpec((16, 256), lambda i: (0, 0)),
                  pl.BlockSpec((1, 256), lambda i: (0, 0)),
                  pl.BlockSpec((1, 256), lambda i: (0, 0)),
                  pl.BlockSpec((256, 128), lambda i: (0, 0)),
                  pl.BlockSpec((1, 128), lambda i: (0, 0)),
                  pl.BlockSpec((1, 128), lambda i: (0, 0)),
                  pl.BlockSpec((1, 128), lambda i: (0, 0))],
        out_specs=[pl.BlockSpec((_MB, 128), lambda i: (i, 0)),
                   pl.BlockSpec((1, 128), lambda i: (0, 0))],
        out_shape=[jax.ShapeDtypeStruct((_N, 128), jnp.float32),
                   jax.ShapeDtypeStruct((1, 128), jnp.float32)],
    )(h2, h2, aug_noise, c, bi, d1h, d1n, db1.reshape(1, -1),
      a1.reshape(1, -1), D2, db2.reshape(1, -1), a2.reshape(1, -1),
      w.reshape(1, -1))
    return ah, stats


def _negsum_body(t_ref, o_ref):
    i = pl.program_id(0)

    @pl.when(i == 0)
    def _():
        o_ref[...] = jnp.zeros_like(o_ref)

    blk = t_ref[...]
    d = jax.lax.broadcasted_iota(jnp.int32, (128, 8), 0)
    g = jax.lax.broadcasted_iota(jnp.int32, (128, 8), 1)
    gmat = (d // 16 == g).astype(jnp.float32)
    q = jnp.dot(blk, gmat, preferred_element_type=jnp.float32)
    o_ref[...] += jnp.zeros((1, 128), jnp.float32) + jnp.sum(_softplus(q))


def _negsum(t):
    return pl.pallas_call(
        _negsum_body,
        grid=(25,),
        in_specs=[pl.BlockSpec((1000, 128), lambda i: (i, 0))],
        out_specs=pl.BlockSpec((1, 128), lambda i: (0, 0)),
        out_shape=jax.ShapeDtypeStruct((1, 128), jnp.float32),
    )(t)


def kernel(x, edge_index, edge_weight, corp_x, corp_edge_index, corp_edge_weight,
           negative_index, aug_noise, W1, b1, W2, b2, D1, db1, a1, D2, db2, a2,
           bi_weights, siamese_w):
    src = edge_index[0].astype(jnp.int32)
    dst = edge_index[1].astype(jnp.int32)
    csrc = corp_edge_index[0].astype(jnp.int32)
    cdst = corp_edge_index[1].astype(jnp.int32)
    pad = _EARR - _E
    src1 = jnp.pad(src, (0, pad)).reshape(_EARR // 128, 128)
    dst1 = jnp.pad(dst, (0, pad)).reshape(_EARR // 128, 128)
    ew1 = jnp.broadcast_to(jnp.pad(edge_weight, (0, pad))[:, None],
                           (_EARR, 16))
    src2 = jnp.pad(csrc + _N, (0, pad)).reshape(_EARR // 128, 128)
    dst2 = jnp.pad(cdst, (0, pad)).reshape(_EARR // 128, 128)
    ew2 = jnp.broadcast_to(jnp.pad(corp_edge_weight, (0, pad))[:, None],
                           (_EARR, 16))

    zeros_hbm = jnp.zeros((625, 64), jnp.float32)
    xs = jnp.concatenate([x, corp_x], axis=0)
    xw = _mm_split(xs, W1)
    ag = _seg1(src1, dst1, ew1, src2, dst2, ew2, xw, zeros_hbm)
    hw = _mm_relu2(ag, b1, W2)
    sg2 = _seg2(src1, dst1, ew1, src2, dst2, ew2, hw, zeros_hbm)
    h2 = _final_h(sg2, b2)

    c = _readout(h2)
    ah, stats = _tail(h2, aug_noise, c, bi_weights,
                      D1, db1, a1, D2, db2, a2, siamese_w[0])
    s_h, s_corp, s_ah, s_pos = stats[0, 0], stats[0, 1], stats[0, 2], stats[0, 3]

    nidx3 = negative_index.astype(jnp.int32).reshape(32, 50, 125)
    t = _neg(h2, ah, nidx3, siamese_w[0])
    s_neg = _negsum(t.reshape(25000, 128))[0, 0]

    dgi = (s_h + s_corp) / (2.0 * _N)
    aug_dgi = _AUG * 1e-05 * (s_ah + s_corp) / (2.0 * _N)
    siam = 1e-05 * (_AUG * 10.0 * s_pos + s_neg) / ((_AUG + _AUG * _NEG) * _N)
    return dgi + aug_dgi + siam


# speedup vs baseline: 2.0884x; 1.0695x over previous
import functools
import jax
import jax.numpy as jnp
from jax import lax
from jax.experimental import pallas as pl
from jax.experimental.pallas import tpu as pltpu
from jax.experimental.pallas import tpu_sc as plsc

_N = 10000
_E = 160000
_EARR = 164096
_MB = 2000
_AUG = 4
_NEG = 5


def _softplus(z):
    return jnp.log(1.0 + jnp.exp(-jnp.abs(z))) + jnp.maximum(z, 0.0)


def _mesh():
    return plsc.VectorSubcoreMesh(core_axis_name="c", subcore_axis_name="s")


def _make_seg(q_groups):
    chunks = 80

    nq2 = q_groups // 2

    @functools.partial(
        pl.kernel,
        out_type=jax.ShapeDtypeStruct((q_groups * 2 * _N, 64), jnp.float32),
        mesh=_mesh(),
        compiler_params=pltpu.CompilerParams(use_tc_tiling_on_sc=False),
        scratch_types=[
            pltpu.VMEM((chunks, 128), jnp.int32),
            pltpu.VMEM((chunks, 128), jnp.int32),
            pltpu.VMEM((128, 16), jnp.float32),
            pltpu.VMEM((128, 16), jnp.float32),
            pltpu.VMEM((128, 64), jnp.float32),
            pltpu.VMEM((128, 64), jnp.float32),
            pltpu.VMEM_SHARED((_N, 64), jnp.float32),
            pltpu.SemaphoreType.DMA,
            pltpu.SemaphoreType.DMA,
            pltpu.SemaphoreType.DMA,
            pltpu.SemaphoreType.DMA,
        ],
    )
    def seg(srcq, dstq, ewq, table, zeros_hbm, out,
            src_a, dst_a, ewb0, ewb1, rows0, rows1, acc,
            gsem0, gsem1, ssem0, ssem1):
        c = lax.axis_index("c")
        s = lax.axis_index("s")
        nrows = _EARR // 128

        def scale(rows_v, ewb):
            for r in range(128):
                wspl = ewb[r]
                for cb in range(4):
                    rows_v[r, pl.ds(cb * 16, 16)] = (
                        rows_v[r, pl.ds(cb * 16, 16)] * wspl)

        def phase(ph, carry):
            g = ph // nq2
            qq = ph - g * nq2
            q = c * nq2 + qq
            v = g * q_groups + q
            sbase = v * nrows + s * chunks
            dbase = g * nrows + s * chunks
            ebase = g * _EARR + s * chunks * 128

            pltpu.sync_copy(zeros_hbm, acc.at[pl.ds(s * 625, 625)])
            pltpu.sync_copy(srcq.at[pl.ds(sbase, chunks)], src_a)
            pltpu.sync_copy(dstq.at[pl.ds(dbase, chunks)], dst_a)
            plsc.subcore_barrier()

            pltpu.sync_copy(ewq.at[pl.ds(ebase, 128)], ewb0)
            pltpu.async_copy(table.at[src_a.at[0]], rows0, gsem0)

            def pipe(j2, carry2):
                a = j2 * 2
                pltpu.sync_copy(ewq.at[pl.ds(ebase + (a + 1) * 128, 128)],
                                ewb1)

                @pl.when(j2 > 0)
                def _():
                    pltpu.make_async_copy(rows1, acc.at[dst_a.at[a - 1]],
                                          ssem1).wait()

                pltpu.async_copy(table.at[src_a.at[a + 1]], rows1, gsem1)
                pltpu.make_async_copy(table.at[src_a.at[a]], rows0,
                                      gsem0).wait()
                scale(rows0, ewb0)
                pltpu.async_copy(rows0, acc.at[dst_a.at[a]], ssem0, add=True)

                @pl.when(a + 2 < chunks)
                def _():
                    pltpu.sync_copy(ewq.at[pl.ds(ebase + (a + 2) * 128, 128)],
                                    ewb0)

                pltpu.make_async_copy(table.at[src_a.at[a + 1]], rows1,
                                      gsem1).wait()
                scale(rows1, ewb1)
                pltpu.async_copy(rows1, acc.at[dst_a.at[a + 1]], ssem1,
                                 add=True)

                @pl.when(a + 2 < chunks)
                def _():
                    pltpu.make_async_copy(rows0, acc.at[dst_a.at[a]],
                                          ssem0).wait()
                    pltpu.async_copy(table.at[src_a.at[a + 2]], rows0, gsem0)

                return carry2

            lax.fori_loop(0, chunks // 2, pipe, 0)
            pltpu.make_async_copy(rows0, acc.at[dst_a.at[chunks - 2]],
                                  ssem0).wait()
            pltpu.make_async_copy(rows1, acc.at[dst_a.at[chunks - 1]],
                                  ssem1).wait()
            plsc.subcore_barrier()
            obase = q * 2 * _N + g * _N
            pltpu.sync_copy(acc.at[pl.ds(s * 625, 625)],
                            out.at[pl.ds(obase + s * 625, 625)])
            plsc.subcore_barrier()
            return carry

        lax.fori_loop(0, 2 * nq2, phase, 0)

    return seg


_seg1 = _make_seg(4)
_seg2 = _make_seg(2)


@functools.partial(
    pl.kernel,
    out_type=jax.ShapeDtypeStruct((1600, 125, 16), jnp.float32),
    mesh=_mesh(),
    scratch_types=[
        pltpu.VMEM((50, 125), jnp.int32),
        pltpu.VMEM((125, 128), jnp.float32),
        pltpu.VMEM((125, 128), jnp.float32),
        pltpu.VMEM((32, 128), jnp.float32),
        pltpu.VMEM((32, 128), jnp.float32),
        pltpu.VMEM((128,), jnp.float32),
        pltpu.VMEM((125, 16), jnp.float32),
        pltpu.VMEM((125, 16), jnp.float32),
        pltpu.SemaphoreType.DMA,
        pltpu.SemaphoreType.DMA,
    ],
)
def _neg(h_tab, ah, nidx3, w_hbm, out, idx_v, neg0, neg1, ah0, ah1, w_v,
         ob0, ob1, sem0, sem1):
    c = lax.axis_index("c")
    s = lax.axis_index("s")
    wid = c * 16 + s
    pltpu.sync_copy(w_hbm, w_v)
    pltpu.sync_copy(nidx3.at[wid], idx_v)
    kbase = wid * 1250

    def load_ah(j, ahbuf):
        i0 = lax.rem(kbase + j * 25, _N)
        i0a = (i0 // 8) * 8
        pltpu.sync_copy(ah.at[pl.ds(i0a, 32)], ahbuf)
        return i0 - i0a

    def compute(j, negbuf, ahbuf, obuf, off):
        def kbody(kl, carry2):
            a = [ahbuf[kl + off, pl.ds(cb * 16, 16)] for cb in range(8)]
            wv = [w_v[pl.ds(cb * 16, 16)] for cb in range(8)]
            for jn in range(_NEG):
                r = kl * _NEG + jn
                t = jnp.zeros((16,), jnp.float32)
                for cb in range(8):
                    t = t + jnp.abs(negbuf[r, pl.ds(cb * 16, 16)] - a[cb]) * wv[cb]
                obuf[r] = t
            return carry2

        lax.fori_loop(0, 25, kbody, 0)
        pltpu.sync_copy(obuf, out.at[wid * 50 + j])

    load_ah(0, ah0)
    pltpu.async_copy(h_tab.at[idx_v.at[0]], neg0, sem0)

    def pipe(j2, carry):
        a = j2 * 2
        off0 = lax.rem(lax.rem(kbase + a * 25, _N), 8)
        off1 = load_ah(a + 1, ah1)
        pltpu.async_copy(h_tab.at[idx_v.at[a + 1]], neg1, sem1)
        pltpu.make_async_copy(h_tab.at[idx_v.at[a]], neg0, sem0).wait()
        compute(a, neg0, ah0, ob0, off0)

        @pl.when(a + 2 < 50)
        def _():
            load_ah(a + 2, ah0)
            pltpu.async_copy(h_tab.at[idx_v.at[a + 2]], neg0, sem0)

        pltpu.make_async_copy(h_tab.at[idx_v.at[a + 1]], neg1, sem1).wait()
        compute(a + 1, neg1, ah1, ob1, off1)
        return carry

    lax.fori_loop(0, 25, pipe, 0)


def _mm_split_body(a_ref, w_ref, o_ref):
    o_ref[...] = jnp.dot(a_ref[...], w_ref[0],
                         preferred_element_type=jnp.float32)


def _mm_split(a, w):
    m = a.shape[0]
    nb = m // _MB
    wr = w.reshape(256, 4, 64).transpose(1, 0, 2)
    return pl.pallas_call(
        _mm_split_body,
        grid=(nb, 4),
        in_specs=[pl.BlockSpec((_MB, 256), lambda i, jc: (i, 0)),
                  pl.BlockSpec((1, 256, 64), lambda i, jc: (jc, 0, 0))],
        out_specs=pl.BlockSpec((_MB, 64), lambda i, jc: (jc * nb + i, 0)),
        out_shape=jax.ShapeDtypeStruct((4 * m, 64), jnp.float32),
    )(a, wr)


def _mm_relu2_body(a0_ref, a1_ref, a2_ref, a3_ref, b_ref, w_ref, o_ref):
    acc = jnp.zeros((_MB, 64), jnp.float32)
    for qq, aref in enumerate((a0_ref, a1_ref, a2_ref, a3_ref)):
        hq = jnp.maximum(aref[...] + b_ref[qq, :][None, :], 0.0)
        acc = acc + jnp.dot(hq, w_ref[0, qq * 64:(qq + 1) * 64, :],
                            preferred_element_type=jnp.float32)
    o_ref[...] = acc


def _mm_relu2(ag, b1, w2):
    q = lambda qq: pl.BlockSpec(
        (_MB, 64), lambda i, g, hc, _qq=qq: (_qq * 10 + g * 5 + i, 0))
    w2r = w2.reshape(256, 2, 64).transpose(1, 0, 2)
    return pl.pallas_call(
        _mm_relu2_body,
        grid=(5, 2, 2),
        in_specs=[q(0), q(1), q(2), q(3),
                  pl.BlockSpec((4, 64), lambda i, g, hc: (0, 0)),
                  pl.BlockSpec((1, 256, 64), lambda i, g, hc: (hc, 0, 0))],
        out_specs=pl.BlockSpec((_MB, 64),
                               lambda i, g, hc: (hc * 10 + g * 5 + i, 0)),
        out_shape=jax.ShapeDtypeStruct((4 * _N, 64), jnp.float32),
    )(ag, ag, ag, ag, b1.reshape(4, 64), w2r)


def _final_h_body(l_ref, r_ref, b_ref, o_ref):
    hl = jnp.maximum(l_ref[...] + b_ref[0, :][None, :], 0.0)
    hr = jnp.maximum(r_ref[...] + b_ref[1, :][None, :], 0.0)
    o_ref[...] = jnp.concatenate([hl, hr], axis=1)


def _final_h(sg2, b2):
    return pl.pallas_call(
        _final_h_body,
        grid=(5, 2),
        in_specs=[pl.BlockSpec((_MB, 64), lambda i, g: (g * 5 + i, 0)),
                  pl.BlockSpec((_MB, 64), lambda i, g: (10 + g * 5 + i, 0)),
                  pl.BlockSpec((2, 64), lambda i, g: (0, 0))],
        out_specs=pl.BlockSpec((_MB, 128), lambda i, g: (g * 5 + i, 0)),
        out_shape=jax.ShapeDtypeStruct((2 * _N, 128), jnp.float32),
    )(sg2, sg2, b2.reshape(2, 64))


def _colsum_body(h_ref, o_ref):
    i = pl.program_id(0)

    @pl.when(i == 0)
    def _():
        o_ref[...] = jnp.zeros_like(o_ref)

    o_ref[...] += jnp.sum(h_ref[...], axis=0, keepdims=True)

    @pl.when(i == pl.num_programs(0) - 1)
    def _():
        o_ref[...] = jax.nn.sigmoid(o_ref[...] / _N)


def _readout(h2):
    return pl.pallas_call(
        _colsum_body,
        grid=(5,),
        in_specs=[pl.BlockSpec((_MB, 128), lambda i: (i, 0))],
        out_specs=pl.BlockSpec((1, 128), lambda i: (0, 0)),
        out_shape=jax.ShapeDtypeStruct((1, 128), jnp.float32),
    )(h2)


def _tail_body(h_ref, ch_ref, an_ref, c_ref, bi_ref, d1h_ref, d1n_ref,
               db1_ref, a1_ref, d2_ref, db2_ref, a2_ref, w_ref,
               ah_ref, s_ref):
    i = pl.program_id(0)
    v = jnp.dot(bi_ref[...], c_ref[...].T,
                preferred_element_type=jnp.float32)
    h = h_ref[...]
    s_h = jnp.sum(_softplus(-jnp.dot(h, v, preferred_element_type=jnp.float32)))
    s_corp = jnp.sum(_softplus(jnp.dot(ch_ref[...], v,
                                       preferred_element_type=jnp.float32)))
    z = (jnp.dot(h, d1h_ref[...], preferred_element_type=jnp.float32)
         + jnp.dot(an_ref[...], d1n_ref[...], preferred_element_type=jnp.float32)
         + db1_ref[...])
    z = jnp.maximum(z, 0.0) + a1_ref[...] * jnp.minimum(z, 0.0)
    ah = jnp.dot(z, d2_ref[...], preferred_element_type=jnp.float32) + db2_ref[...]
    ah = jnp.maximum(ah, 0.0) + a2_ref[...] * jnp.minimum(ah, 0.0)
    ah_ref[...] = ah
    s_ah = jnp.sum(_softplus(-jnp.dot(ah, v, preferred_element_type=jnp.float32)))
    p = jnp.sum(jnp.abs(ah - h) * w_ref[...], axis=1)
    s_pos = jnp.sum(_softplus(-p))

    @pl.when(i == 0)
    def _():
        s_ref[...] = jnp.zeros_like(s_ref)

    lane = jax.lax.broadcasted_iota(jnp.int32, (1, 128), 1)
    s_ref[...] += (jnp.where(lane == 0, s_h, 0.0)
                   + jnp.where(lane == 1, s_corp, 0.0)
                   + jnp.where(lane == 2, s_ah, 0.0)
                   + jnp.where(lane == 3, s_pos, 0.0))


def _tail(h2, aug_noise, c, bi, D1, db1, a1, D2, db2, a2, w):
    d1h = D1[:128]
    d1n = D1[128:]
    ah, stats = pl.pallas_call(
        _tail_body,
        grid=(5,),
        in_specs=[pl.BlockSpec((_MB, 128), lambda i: (i, 0)),
                  pl.BlockSpec((_MB, 128), lambda i: (i + 5, 0)),
                  pl.BlockSpec((_MB, 16), lambda i: (i, 0)),
                  pl.BlockSpec((1, 128), lambda i: (0, 0)),
                  pl.BlockSpec((128, 128), lambda i: (0, 0)),
                  pl.BlockSpec((128, 256), lambda i: (0, 0)),
                  pl.BlockSpec((16, 256), lambda i: (0, 0)),
                  pl.BlockSpec((1, 256), lambda i: (0, 0)),
                  pl.BlockSpec((1, 256), lambda i: (0, 0)),
                  pl.BlockSpec((256, 128), lambda i: (0, 0)),
                  pl.BlockSpec((1, 128), lambda i: (0, 0)),
                  pl.BlockSpec((1, 128), lambda i: (0, 0)),
                  pl.BlockSpec((1, 128), lambda i: (0, 0))],
        out_specs=[pl.BlockSpec((_MB, 128), lambda i: (i, 0)),
                   pl.BlockSpec((1, 128), lambda i: (0, 0))],
        out_shape=[jax.ShapeDtypeStruct((_N, 128), jnp.float32),
                   jax.ShapeDtypeStruct((1, 128), jnp.float32)],
    )(h2, h2, aug_noise, c, bi, d1h, d1n, db1.reshape(1, -1),
      a1.reshape(1, -1), D2, db2.reshape(1, -1), a2.reshape(1, -1),
      w.reshape(1, -1))
    return ah, stats


def _negsum_body(t_ref, o_ref):
    i = pl.program_id(0)

    @pl.when(i == 0)
    def _():
        o_ref[...] = jnp.zeros_like(o_ref)

    blk = t_ref[...]
    d = jax.lax.broadcasted_iota(jnp.int32, (128, 8), 0)
    g = jax.lax.broadcasted_iota(jnp.int32, (128, 8), 1)
    gmat = (d // 16 == g).astype(jnp.float32)
    q = jnp.dot(blk, gmat, preferred_element_type=jnp.float32)
    o_ref[...] += jnp.zeros((1, 128), jnp.float32) + jnp.sum(_softplus(q))


def _negsum(t):
    return pl.pallas_call(
        _negsum_body,
        grid=(25,),
        in_specs=[pl.BlockSpec((1000, 128), lambda i: (i, 0))],
        out_specs=pl.BlockSpec((1, 128), lambda i: (0, 0)),
        out_shape=jax.ShapeDtypeStruct((1, 128), jnp.float32),
    )(t)


def kernel(x, edge_index, edge_weight, corp_x, corp_edge_index, corp_edge_weight,
           negative_index, aug_noise, W1, b1, W2, b2, D1, db1, a1, D2, db2, a2,
           bi_weights, siamese_w):
    src = edge_index[0].astype(jnp.int32)
    dst = edge_index[1].astype(jnp.int32)
    csrc = corp_edge_index[0].astype(jnp.int32)
    cdst = corp_edge_index[1].astype(jnp.int32)
    pad = _EARR - _E
    nrows = _EARR // 128
    sp1 = jnp.pad(src, (0, pad))
    sp2 = jnp.pad(csrc + _N, (0, pad))
    qoff = (jnp.arange(4, dtype=jnp.int32) * (2 * _N))[:, None]
    srcq1 = jnp.concatenate(
        [(sp1[None, :] + qoff).reshape(4 * nrows, 128),
         (sp2[None, :] + qoff).reshape(4 * nrows, 128)], axis=0)
    srcq2 = jnp.concatenate(
        [(sp1[None, :] + qoff[:2]).reshape(2 * nrows, 128),
         (sp2[None, :] + qoff[:2]).reshape(2 * nrows, 128)], axis=0)
    dstq = jnp.concatenate([jnp.pad(dst, (0, pad)).reshape(nrows, 128),
                            jnp.pad(cdst, (0, pad)).reshape(nrows, 128)],
                           axis=0)
    ewq = jnp.concatenate(
        [jnp.broadcast_to(jnp.pad(edge_weight, (0, pad))[:, None],
                          (_EARR, 16)),
         jnp.broadcast_to(jnp.pad(corp_edge_weight, (0, pad))[:, None],
                          (_EARR, 16))], axis=0)

    zeros_hbm = jnp.zeros((625, 64), jnp.float32)
    xs = jnp.concatenate([x, corp_x], axis=0)
    xw = _mm_split(xs, W1)
    ag = _seg1(srcq1, dstq, ewq, xw, zeros_hbm)
    hw = _mm_relu2(ag, b1, W2)
    sg2 = _seg2(srcq2, dstq, ewq, hw, zeros_hbm)
    h2 = _final_h(sg2, b2)

    c = _readout(h2)
    ah, stats = _tail(h2, aug_noise, c, bi_weights,
                      D1, db1, a1, D2, db2, a2, siamese_w[0])
    s_h, s_corp, s_ah, s_pos = stats[0, 0], stats[0, 1], stats[0, 2], stats[0, 3]

    nidx3 = negative_index.astype(jnp.int32).reshape(32, 50, 125)
    t = _neg(h2, ah, nidx3, siamese_w[0])
    s_neg = _negsum(t.reshape(25000, 128))[0, 0]

    dgi = (s_h + s_corp) / (2.0 * _N)
    aug_dgi = _AUG * 1e-05 * (s_ah + s_corp) / (2.0 * _N)
    siam = 1e-05 * (_AUG * 10.0 * s_pos + s_neg) / ((_AUG + _AUG * _NEG) * _N)
    return dgi + aug_dgi + siam
